# baseline (speedup 1.0000x reference)
import functools

import jax
import jax.numpy as jnp
from jax import lax
from jax.experimental import pallas as pl
from jax.experimental.pallas import tpu as pltpu
from jax.experimental.pallas import tpu_sc as plsc

N = 10000
E = 320000
D = 128
C = 10

NC = 2
NS = 16
CH = 72
NCH = 139
NCHP = 144
NRING = 5
NPAD = 10112
NPT = NPAD // NS

_mesh = plsc.VectorSubcoreMesh(core_axis_name="c", subcore_axis_name="s")


@functools.partial(
    pl.kernel,
    out_type=jax.ShapeDtypeStruct((NC, NPAD, D), jnp.float32),
    mesh=_mesh,
    scratch_types=[
        [pltpu.VMEM((CH, D), jnp.float32) for _ in range(NRING)],
        [pltpu.VMEM((CH,), jnp.int32) for _ in range(NRING)],
        [pltpu.VMEM((CH,), jnp.int32) for _ in range(NRING)],
        pltpu.VMEM_SHARED((NPAD, D), jnp.float32),
        [pltpu.SemaphoreType.DMA for _ in range(NRING)],
        [pltpu.SemaphoreType.DMA for _ in range(NRING)],
        [pltpu.SemaphoreType.DMA for _ in range(NRING)],
        [pltpu.SemaphoreType.DMA for _ in range(NRING)],
    ],
)
def _agg(h_hbm, src_hbm, dst_hbm, zeros_hbm, out_hbm,
         rowb, srcb, dstb, acc, semg, sema, sems, semd):
    c = lax.axis_index("c")
    s = lax.axis_index("s")
    G = 2
    pltpu.sync_copy(zeros_hbm, acc.at[pl.ds(s * NPT, NPT)])
    plsc.subcore_barrier()

    def load_src(j, b):
        pltpu.async_copy(src_hbm.at[c, s, j], srcb[b], sems[b])

    def load_dst(j, b):
        pltpu.async_copy(dst_hbm.at[c, s, j], dstb[b], semd[b])

    def chunk(j, b, bg, *, first=False, gat=True, ldsrc=True):
        if not first:
            pltpu.make_async_copy(rowb[bg], acc.at[dstb[bg]],
                                  sema[bg]).wait()
        if gat:
            pltpu.make_async_copy(src_hbm.at[c, s, 0], srcb[bg],
                                  sems[bg]).wait()
            pltpu.async_copy(h_hbm.at[srcb[bg]], rowb[bg], semg[bg])
            load_dst(j + G, bg)
        pltpu.make_async_copy(h_hbm.at[srcb[b]], rowb[b], semg[b]).wait()
        if ldsrc:
            load_src(j + NRING, b)
        pltpu.make_async_copy(dst_hbm.at[c, s, 0], dstb[b], semd[b]).wait()
        pltpu.async_copy(rowb[b], acc.at[dstb[b]], sema[b], add=True)

    for b in range(NRING):
        load_src(b, b)
    for b in range(G):
        load_dst(b, b)
        pltpu.make_async_copy(src_hbm.at[c, s, 0], srcb[b], sems[b]).wait()
        pltpu.async_copy(h_hbm.at[srcb[b]], rowb[b], semg[b])

    chunk(0, 0, 2, first=True)
    chunk(1, 1, 3, first=True)
    chunk(2, 2, 4, first=True)

    @pl.loop(3, 133, step=NRING)
    def _(base):
        for off in range(NRING):
            chunk(base + off, (3 + off) % NRING, (3 + off + G) % NRING)

    for j in range(133, NCH):
        chunk(j, j % NRING, (j + G) % NRING,
              gat=(j + G <= NCH - 1), ldsrc=(j + NRING <= NCH - 1))

    for j in range(NCH - (NRING - G), NCH):
        b = j % NRING
        pltpu.make_async_copy(rowb[b], acc.at[dstb[b]], sema[b]).wait()

    plsc.subcore_barrier()
    pltpu.sync_copy(acc.at[pl.ds(s * NPT, NPT)],
                    out_hbm.at[c].at[pl.ds(s * NPT, NPT)])


BR = 1000
_GRID = N // BR


def _dot_t(x, w):
    return lax.dot_general(x, w, (((1,), (1,)), ((), ())),
                           preferred_element_type=jnp.float32,
                           precision=lax.Precision.HIGHEST)


def _mlp_body(eps_ref, h_ref, a0_ref, a1_ref, w1_ref, b1_ref, w2_ref, b2_ref,
              o_ref):
    z = (1.0 + eps_ref[0]) * h_ref[...] + a0_ref[0] + a1_ref[0]
    z = jnp.maximum(_dot_t(z, w1_ref[...]) + b1_ref[...], 0.0)
    z = _dot_t(z, w2_ref[...]) + b2_ref[...]
    o_ref[...] = jnp.maximum(z, 0.0)


def _mlp(h, agg, eps, W1, b1, W2, b2):
    full = lambda shape: pl.BlockSpec(shape, lambda i: (0,) * len(shape))
    row = pl.BlockSpec((BR, D), lambda i: (i, 0))
    a0 = pl.BlockSpec((1, BR, D), lambda i: (0, i, 0))
    a1 = pl.BlockSpec((1, BR, D), lambda i: (1, i, 0))
    return pl.pallas_call(
        _mlp_body,
        grid=(_GRID,),
        in_specs=[
            pl.BlockSpec(memory_space=pltpu.SMEM),
            row, a0, a1,
            full((D, D)), full((1, D)), full((D, D)), full((1, D)),
        ],
        out_specs=row,
        out_shape=jax.ShapeDtypeStruct((N, D), jnp.float32),
    )(eps.reshape(1), h, agg, agg, W1, b1.reshape(1, D), W2, b2.reshape(1, D))


def _final_body(eps_ref, h_ref, a0_ref, a1_ref, w1_ref, b1_ref, w2_ref,
                b2_ref, wc1_ref, bc1_ref, wc2_ref, bc2_ref, o_ref, acc_ref):
    i = pl.program_id(0)
    z = (1.0 + eps_ref[0]) * h_ref[...] + a0_ref[0] + a1_ref[0]
    z = jnp.maximum(_dot_t(z, w1_ref[...]) + b1_ref[...], 0.0)
    z = _dot_t(z, w2_ref[...]) + b2_ref[...]
    h3 = jnp.maximum(z, 0.0)
    part = jnp.sum(h3, axis=0, keepdims=True)

    @pl.when(i == 0)
    def _():
        acc_ref[...] = jnp.zeros_like(acc_ref)

    acc_ref[...] += part

    @pl.when(i == pl.num_programs(0) - 1)
    def _():
        hg = acc_ref[...]
        t = jnp.maximum(_dot_t(hg, wc1_ref[...]) + bc1_ref[...], 0.0)
        o_ref[...] = _dot_t(t, wc2_ref[...]) + bc2_ref[...]


def _final(h, agg, eps, W1, b1, W2, b2, Wc1, bc1, Wc2, bc2):
    full = lambda shape: pl.BlockSpec(shape, lambda i: (0,) * len(shape))
    row = pl.BlockSpec((BR, D), lambda i: (i, 0))
    a0 = pl.BlockSpec((1, BR, D), lambda i: (0, i, 0))
    a1 = pl.BlockSpec((1, BR, D), lambda i: (1, i, 0))
    return pl.pallas_call(
        _final_body,
        grid=(_GRID,),
        in_specs=[
            pl.BlockSpec(memory_space=pltpu.SMEM),
            row, a0, a1,
            full((D, D)), full((1, D)), full((D, D)), full((1, D)),
            full((D, D)), full((1, D)), full((C, D)), full((1, C)),
        ],
        out_specs=full((1, C)),
        out_shape=jax.ShapeDtypeStruct((1, C), jnp.float32),
        scratch_shapes=[pltpu.VMEM((1, D), jnp.float32)],
    )(eps.reshape(1), h, agg, agg, W1, b1.reshape(1, D), W2, b2.reshape(1, D),
      Wc1, bc1.reshape(1, D), Wc2, bc2.reshape(1, C))


def kernel(features, edge_index,
           eps0, W1_0, b1_0, W2_0, b2_0,
           eps1, W1_1, b1_1, W2_1, b2_1,
           eps2, W1_2, b1_2, W2_2, b2_2,
           Wc1, bc1, Wc2, bc2):
    pad = NCHP * CH - E // (NC * NS)
    src = jnp.pad(edge_index[0].reshape(NC * NS, -1), ((0, 0), (0, pad)),
                  constant_values=0).reshape(NC, NS, NCHP, CH)
    padvals = N + (jnp.arange(pad, dtype=jnp.int32) % (NPAD - N))
    dst = jnp.concatenate(
        [edge_index[1].reshape(NC * NS, -1),
         jnp.broadcast_to(padvals, (NC * NS, pad))],
        axis=1).reshape(NC, NS, NCHP, CH)
    zeros = jnp.zeros((NPT, D), jnp.float32)
    layers = [
        (eps0, W1_0, b1_0, W2_0, b2_0),
        (eps1, W1_1, b1_1, W2_1, b2_1),
        (eps2, W1_2, b1_2, W2_2, b2_2),
    ]
    h = features
    for li, (eps, W1, b1, W2, b2) in enumerate(layers):
        agg = _agg(h, src, dst, zeros)
        if li < 2:
            h = _mlp(h, agg, eps, W1, b1, W2, b2)
        else:
            out = _final(h, agg, eps, W1, b1, W2, b2,
                         Wc1, bc1, Wc2, bc2)
    return out


# speedup vs baseline: 1.2785x; 1.2785x over previous
import functools

import jax
import jax.numpy as jnp
from jax import lax
from jax.experimental import pallas as pl
from jax.experimental.pallas import tpu as pltpu
from jax.experimental.pallas import tpu_sc as plsc

N = 10000
E = 320000
D = 128
C = 10

NC = 2
NS = 16
CH = 80
NCH = 125
NCHP = 127
NPAD = 10112
NPT = NPAD // NS

_mesh = plsc.VectorSubcoreMesh(core_axis_name="c", subcore_axis_name="s")


@functools.partial(
    pl.kernel,
    out_type=jax.ShapeDtypeStruct((NC, NPAD, D), jnp.float32),
    mesh=_mesh,
    scratch_types=[
        [pltpu.VMEM((CH, D), jnp.float32) for _ in range(4)],
        [pltpu.VMEM((CH,), jnp.int32) for _ in range(4)],
        [pltpu.VMEM((CH,), jnp.int32) for _ in range(4)],
        pltpu.VMEM_SHARED((NPAD, D), jnp.float32),
        [pltpu.SemaphoreType.DMA for _ in range(4)],
        [pltpu.SemaphoreType.DMA for _ in range(4)],
        [pltpu.SemaphoreType.DMA for _ in range(4)],
        [pltpu.SemaphoreType.DMA for _ in range(4)],
    ],
)
def _agg(h_hbm, ei_hbm, zeros_hbm, out_hbm,
         rowb, srcb, dstb, acc, semg, sema, sems, semd):
    c = lax.axis_index("c")
    s = lax.axis_index("s")
    base = (c * NS + s) * (E // (NC * NS))
    pltpu.sync_copy(zeros_hbm, acc.at[pl.ds(s * NPT, NPT)])
    plsc.subcore_barrier()

    def load_src(j, b):
        jj = jnp.minimum(j, NCH - 1)
        pltpu.async_copy(ei_hbm.at[pl.ds(base + jj * CH, CH)],
                         srcb[b], sems[b])

    def load_dst(j, b):
        pltpu.async_copy(ei_hbm.at[pl.ds(E + base + j * CH, CH)],
                         dstb[b], semd[b])

    def chunk(j, b, bp2, *, first=False, g2=True, s4=True):
        if not first:
            pltpu.make_async_copy(rowb[bp2], acc.at[dstb[bp2]],
                                  sema[bp2]).wait()
        if g2:
            pltpu.make_async_copy(ei_hbm.at[pl.ds(0, CH)], srcb[bp2],
                                  sems[bp2]).wait()
            pltpu.async_copy(h_hbm.at[srcb[bp2]], rowb[bp2], semg[bp2])
            load_dst(j + 2, bp2)
        pltpu.make_async_copy(h_hbm.at[srcb[b]], rowb[b], semg[b]).wait()
        if s4:
            load_src(j + 4, b)
        pltpu.make_async_copy(ei_hbm.at[pl.ds(0, CH)], dstb[b],
                              semd[b]).wait()
        pltpu.async_copy(rowb[b], acc.at[dstb[b]], sema[b], add=True)

    for b in range(4):
        load_src(b, b)
    load_dst(0, 0)
    load_dst(1, 1)
    for b in range(2):
        pltpu.make_async_copy(ei_hbm.at[pl.ds(0, CH)], srcb[b],
                              sems[b]).wait()
        pltpu.async_copy(h_hbm.at[srcb[b]], rowb[b], semg[b])
    chunk(0, 0, 2, first=True)
    chunk(1, 1, 3, first=True)

    @pl.loop(2, 122, step=4)
    def _(base):
        chunk(base, 2, 0)
        chunk(base + 1, 3, 1)
        chunk(base + 2, 0, 2)
        chunk(base + 3, 1, 3)

    chunk(122, 2, 0, s4=False)
    chunk(123, 3, 1, g2=False, s4=False)
    chunk(124, 0, 2, g2=False, s4=False)
    pltpu.make_async_copy(rowb[3], acc.at[dstb[3]], sema[3]).wait()
    pltpu.make_async_copy(rowb[0], acc.at[dstb[0]], sema[0]).wait()
    pltpu.make_async_copy(ei_hbm.at[pl.ds(0, CH)], srcb[1], sems[1]).wait()

    plsc.subcore_barrier()
    pltpu.sync_copy(acc.at[pl.ds(s * NPT, NPT)],
                    out_hbm.at[c].at[pl.ds(s * NPT, NPT)])


BR = 2000
_GRID = N // BR


def _dot_t(x, w):
    return lax.dot_general(x, w, (((1,), (1,)), ((), ())),
                           preferred_element_type=jnp.float32)


def _mlp_body(eps_ref, h_ref, a0_ref, a1_ref, w1_ref, b1_ref, w2_ref, b2_ref,
              o_ref):
    z = (1.0 + eps_ref[0]) * h_ref[...] + a0_ref[0] + a1_ref[0]
    z = jnp.maximum(_dot_t(z, w1_ref[...]) + b1_ref[...], 0.0)
    z = _dot_t(z, w2_ref[...]) + b2_ref[...]
    o_ref[...] = jnp.maximum(z, 0.0)


def _mlp(h, agg, eps, W1, b1, W2, b2):
    full = lambda shape: pl.BlockSpec(shape, lambda i: (0,) * len(shape))
    row = pl.BlockSpec((BR, D), lambda i: (i, 0))
    a0 = pl.BlockSpec((1, BR, D), lambda i: (0, i, 0))
    a1 = pl.BlockSpec((1, BR, D), lambda i: (1, i, 0))
    return pl.pallas_call(
        _mlp_body,
        grid=(_GRID,),
        in_specs=[
            pl.BlockSpec(memory_space=pltpu.SMEM),
            row, a0, a1,
            full((D, D)), full((1, D)), full((D, D)), full((1, D)),
        ],
        out_specs=row,
        out_shape=jax.ShapeDtypeStruct((N, D), jnp.float32),
    )(eps.reshape(1), h, agg, agg, W1, b1.reshape(1, D), W2, b2.reshape(1, D))


def _final_body(eps_ref, h_ref, a0_ref, a1_ref, w1_ref, b1_ref, w2_ref,
                b2_ref, wc1_ref, bc1_ref, wc2_ref, bc2_ref, o_ref, acc_ref):
    i = pl.program_id(0)
    z = (1.0 + eps_ref[0]) * h_ref[...] + a0_ref[0] + a1_ref[0]
    z = jnp.maximum(_dot_t(z, w1_ref[...]) + b1_ref[...], 0.0)
    z = _dot_t(z, w2_ref[...]) + b2_ref[...]
    h3 = jnp.maximum(z, 0.0)
    part = jnp.sum(h3, axis=0, keepdims=True)

    @pl.when(i == 0)
    def _():
        acc_ref[...] = jnp.zeros_like(acc_ref)

    acc_ref[...] += part

    @pl.when(i == pl.num_programs(0) - 1)
    def _():
        hg = acc_ref[...]
        t = jnp.maximum(_dot_t(hg, wc1_ref[...]) + bc1_ref[...], 0.0)
        o_ref[...] = _dot_t(t, wc2_ref[...]) + bc2_ref[...]


def _final(h, agg, eps, W1, b1, W2, b2, Wc1, bc1, Wc2, bc2):
    full = lambda shape: pl.BlockSpec(shape, lambda i: (0,) * len(shape))
    row = pl.BlockSpec((BR, D), lambda i: (i, 0))
    a0 = pl.BlockSpec((1, BR, D), lambda i: (0, i, 0))
    a1 = pl.BlockSpec((1, BR, D), lambda i: (1, i, 0))
    return pl.pallas_call(
        _final_body,
        grid=(_GRID,),
        in_specs=[
            pl.BlockSpec(memory_space=pltpu.SMEM),
            row, a0, a1,
            full((D, D)), full((1, D)), full((D, D)), full((1, D)),
            full((D, D)), full((1, D)), full((C, D)), full((1, C)),
        ],
        out_specs=full((1, C)),
        out_shape=jax.ShapeDtypeStruct((1, C), jnp.float32),
        scratch_shapes=[pltpu.VMEM((1, D), jnp.float32)],
    )(eps.reshape(1), h, agg, agg, W1, b1.reshape(1, D), W2, b2.reshape(1, D),
      Wc1, bc1.reshape(1, D), Wc2, bc2.reshape(1, C))


def kernel(features, edge_index,
           eps0, W1_0, b1_0, W2_0, b2_0,
           eps1, W1_1, b1_1, W2_1, b2_1,
           eps2, W1_2, b1_2, W2_2, b2_2,
           Wc1, bc1, Wc2, bc2):
    zeros = jnp.zeros((NPT, D), jnp.float32)
    layers = [
        (eps0, W1_0, b1_0, W2_0, b2_0),
        (eps1, W1_1, b1_1, W2_1, b2_1),
        (eps2, W1_2, b1_2, W2_2, b2_2),
    ]
    h = features
    for li, (eps, W1, b1, W2, b2) in enumerate(layers):
        agg = _agg(h, edge_index.reshape(2 * E), zeros)
        if li < 2:
            h = _mlp(h, agg, eps, W1, b1, W2, b2)
        else:
            out = _final(h, agg, eps, W1, b1, W2, b2,
                         Wc1, bc1, Wc2, bc2)
    return out
